# Initial kernel scaffold; baseline (speedup 1.0000x reference)
import jax
import jax.numpy as jnp
from jax.experimental import pallas as pl


def kernel(input_user, input_item, src_fwd_0, dst_fwd_0, rid_fwd_0, src_rev_0, dst_rev_0, rid_rev_0, src_fwd_1, dst_fwd_1, rid_fwd_1, src_rev_1, dst_rev_1, rid_rev_1, cj_user, ci_user, cj_item, ci_item, params):
    raise NotImplementedError("write your pallas kernel here")


# speedup vs baseline: 2.5989x; 2.5989x over previous
import functools

import jax
import jax.numpy as jnp
from jax import lax
from jax.experimental import pallas as pl
from jax.experimental.pallas import tpu as pltpu
from jax.experimental.pallas import tpu_sc as plsc

N = 50000
E = 400000
D = 128

NC = 2
NS = 16
NW = NC * NS
CH = 128

EP = ((E + NW * CH - 1) // (NW * CH)) * (NW * CH)
NP = ((N + NW * CH - 1) // (NW * CH)) * (NW * CH)
N1 = 53760
DUMP = N + 1

_MESH = plsc.VectorSubcoreMesh(core_axis_name="c", subcore_axis_name="s")


def _sc_gather(table, idx, d):
  b = idx.shape[0]
  per_w = b // NW
  nch = per_w // CH

  @functools.partial(
      pl.kernel,
      out_type=jax.ShapeDtypeStruct((b, d), jnp.float32),
      mesh=_MESH,
      scratch_types=[
          pltpu.VMEM((CH,), jnp.int32),
          pltpu.VMEM((CH, d), jnp.float32),
          pltpu.SemaphoreType.DMA,
      ],
  )
  def k(table_hbm, idx_hbm, out_hbm, idx_v, rows_v, sem):
    wid = lax.axis_index("s") * NC + lax.axis_index("c")
    base = wid * per_w

    def body(i, carry):
      off = base + i * CH
      pltpu.sync_copy(idx_hbm.at[pl.ds(off, CH)], idx_v)
      pltpu.async_copy(table_hbm.at[idx_v], rows_v, sem).wait()
      pltpu.sync_copy(rows_v, out_hbm.at[pl.ds(off, CH)])
      return carry

    lax.fori_loop(0, nch, body, 0)

  return k(table, idx)


def _sc_scatter_add(m, dst):
  e_per_tile = EP // NS
  nch = e_per_tile // CH
  rng = 8960
  arows = 10240
  ldump = arows - 1

  @functools.partial(
      pl.kernel,
      out_type=jax.ShapeDtypeStruct((N1, D), jnp.float32),
      mesh=_MESH,
      scratch_types=[
          pltpu.VMEM((CH,), jnp.int32),
          pltpu.VMEM((CH, D), jnp.float32),
          pltpu.VMEM((CH, D), jnp.float32),
          pltpu.VMEM_SHARED((arows, D), jnp.float32),
          pltpu.SemaphoreType.DMA,
      ],
  )
  def k(m_hbm, dst_hbm, out_hbm, idx_v, buf_v, zero_v, agg_sh, sem):
    cid = lax.axis_index("c")
    sid = lax.axis_index("s")
    ebase = sid * e_per_tile
    z16 = jnp.zeros((16,), jnp.float32)

    def zbody(r, carry):
      for c0 in range(0, D, 16):
        zero_v[r, c0:c0 + 16] = z16
      return carry

    lax.fori_loop(0, CH, zbody, 0)

    for third in range(3):
      rp = cid * 3 + third
      lo = rp * rng

      def fill(j, carry):
        pltpu.sync_copy(zero_v, agg_sh.at[pl.ds(sid * 640 + j * CH, CH)])
        return carry

      lax.fori_loop(0, 5, fill, 0)
      plsc.subcore_barrier()

      def scat(i, carry):
        off = ebase + i * CH
        pltpu.sync_copy(dst_hbm.at[pl.ds(off, CH)], idx_v)
        pltpu.sync_copy(m_hbm.at[pl.ds(off, CH)], buf_v)
        for kk in range(CH // 16):
          v = idx_v[kk * 16:(kk + 1) * 16]
          local = v - lo
          ok = (local >= 0) & (local < rng)
          idx_v[kk * 16:(kk + 1) * 16] = jnp.where(ok, local, ldump)
        pltpu.sync_copy(buf_v, agg_sh.at[idx_v], add=True)
        return carry

      lax.fori_loop(0, nch, scat, 0)
      plsc.subcore_barrier()

      pltpu.sync_copy(agg_sh.at[pl.ds(sid * 560, 560)],
                      out_hbm.at[pl.ds(lo + sid * 560, 560)])
      plsc.subcore_barrier()

  return k(m, dst)


_NODE_BK = 1000
_EDGE_BK = 512


def _hcj_body(feat_ref, cj_ref, w_ref, b_ref, out_ref):
  h = jnp.dot(feat_ref[...], w_ref[...], preferred_element_type=jnp.float32)
  out_ref[...] = (h + b_ref[...]) * cj_ref[...]


def _tc_hcj(feat, cj, emw_t, emb):
  grid = N // _NODE_BK
  return pl.pallas_call(
      _hcj_body,
      grid=(grid,),
      in_specs=[
          pl.BlockSpec((_NODE_BK, D), lambda i: (i, 0)),
          pl.BlockSpec((_NODE_BK, 1), lambda i: (i, 0)),
          pl.BlockSpec((D, D), lambda i: (0, 0)),
          pl.BlockSpec((1, D), lambda i: (0, 0)),
      ],
      out_specs=pl.BlockSpec((_NODE_BK, D), lambda i: (i, 0)),
      out_shape=jax.ShapeDtypeStruct((N, D), jnp.float32),
  )(feat, cj, emw_t, emb)


def _gelu(x):
  return 0.5 * x * (1.0 + lax.erf(x * 0.7071067811865476))


def _edge_body(rf_ref, hs_ref, cj_ref, pw_ref, rsw_ref,
               w1_ref, w2_ref, w3_ref, out_ref):
  rfeat = rf_ref[...]
  pa = jax.nn.sigmoid(jnp.sum(rfeat * pw_ref[...], axis=1, keepdims=True))
  ra = jax.nn.sigmoid(jnp.sum(rfeat * rsw_ref[...], axis=1, keepdims=True))
  g = _gelu(jnp.dot(rfeat, w1_ref[...], preferred_element_type=jnp.float32))
  g = _gelu(jnp.dot(g, w2_ref[...], preferred_element_type=jnp.float32))
  rf = jnp.dot(g, w3_ref[...], preferred_element_type=jnp.float32)
  cj = cj_ref[...][:, 0:1]
  out_ref[...] = hs_ref[...] * pa + rf * (ra * cj)


def _tc_edge(rfeat, hsrc, cjsrc, pw, rsw, w1_t, w2_t, w3_t):
  grid = EP // _EDGE_BK
  return pl.pallas_call(
      _edge_body,
      grid=(grid,),
      in_specs=[
          pl.BlockSpec((_EDGE_BK, D), lambda i: (i, 0)),
          pl.BlockSpec((_EDGE_BK, D), lambda i: (i, 0)),
          pl.BlockSpec((_EDGE_BK, D), lambda i: (i, 0)),
          pl.BlockSpec((1, D), lambda i: (0, 0)),
          pl.BlockSpec((1, D), lambda i: (0, 0)),
          pl.BlockSpec((D, D), lambda i: (0, 0)),
          pl.BlockSpec((D, D), lambda i: (0, 0)),
          pl.BlockSpec((D, D), lambda i: (0, 0)),
      ],
      out_specs=pl.BlockSpec((_EDGE_BK, D), lambda i: (i, 0)),
      out_shape=jax.ShapeDtypeStruct((EP, D), jnp.float32),
  )(rfeat, hsrc, cjsrc, pw, rsw, w1_t, w2_t, w3_t)


def _final_body(a_ref, b_ref, ci_ref, lwa_ref, lwb_ref, lb_ref,
                fc_ref, fcb_ref, out_ref):
  ci = ci_ref[...]
  s = jnp.dot(a_ref[...] * ci, lwa_ref[...], preferred_element_type=jnp.float32)
  s = s + jnp.dot(b_ref[...] * ci, lwb_ref[...],
                  preferred_element_type=jnp.float32)
  s = s + lb_ref[...]
  out_ref[...] = jnp.dot(_gelu(s), fc_ref[...],
                         preferred_element_type=jnp.float32) + fcb_ref[...]


def _tc_final(agg_a, agg_b, ci, lwa_t, lwb_t, lbsum, fc_t, fcb):
  grid = N // _NODE_BK
  return pl.pallas_call(
      _final_body,
      grid=(grid,),
      in_specs=[
          pl.BlockSpec((_NODE_BK, D), lambda i: (i, 0)),
          pl.BlockSpec((_NODE_BK, D), lambda i: (i, 0)),
          pl.BlockSpec((_NODE_BK, 1), lambda i: (i, 0)),
          pl.BlockSpec((D, D), lambda i: (0, 0)),
          pl.BlockSpec((D, D), lambda i: (0, 0)),
          pl.BlockSpec((1, D), lambda i: (0, 0)),
          pl.BlockSpec((D, D), lambda i: (0, 0)),
          pl.BlockSpec((1, D), lambda i: (0, 0)),
      ],
      out_specs=pl.BlockSpec((_NODE_BK, D), lambda i: (i, 0)),
      out_shape=jax.ShapeDtypeStruct((N, D), jnp.float32),
  )(agg_a, agg_b, ci, lwa_t, lwb_t, lbsum, fc_t, fcb)


def _pad_idx(idx, total, fill):
  return jnp.pad(idx.astype(jnp.int32), (0, total - idx.shape[0]),
                 constant_values=fill)


def kernel(input_user, input_item,
           src_fwd_0, dst_fwd_0, rid_fwd_0, src_rev_0, dst_rev_0, rid_rev_0,
           src_fwd_1, dst_fwd_1, rid_fwd_1, src_rev_1, dst_rev_1, rid_rev_1,
           cj_user, ci_user, cj_item, ci_item, params):
  table = params["table"]
  convs = params["convs"]

  feat_u = _sc_gather(params["uemb"], _pad_idx(input_user, NP, 0), D)
  feat_i = _sc_gather(params["iemb"], _pad_idx(input_item, NP, 0), D)

  cj128_u = jnp.tile(cj_user, (1, D))
  cj128_i = jnp.tile(cj_item, (1, D))

  edges = [
      (src_fwd_0, dst_fwd_0, rid_fwd_0, feat_u, cj_user, cj128_u),
      (src_rev_0, dst_rev_0, rid_rev_0, feat_i, cj_item, cj128_i),
      (src_fwd_1, dst_fwd_1, rid_fwd_1, feat_u, cj_user, cj128_u),
      (src_rev_1, dst_rev_1, rid_rev_1, feat_i, cj_item, cj128_i),
  ]

  aggs = []
  for c, (src, dst, rid, feat, cj, cjw) in enumerate(edges):
    cp = convs[c]
    hcj = _tc_hcj(feat, cj, cp["emw"].T, cp["emb"].reshape(1, D))
    src_p = _pad_idx(src, EP, 0)
    rfeat = _sc_gather(table, _pad_idx(rid, EP, 0), D)
    hsrc = _sc_gather(hcj, src_p, D)
    cjsrc = _sc_gather(cjw, src_p, D)
    m = _tc_edge(rfeat, hsrc, cjsrc,
                 cp["pw"], cp["rsw"],
                 cp["rw1"].T, cp["rw2"].T, cp["rw3"].T)
    agg = _sc_scatter_add(m, _pad_idx(dst, EP, DUMP))
    aggs.append(agg)

  item_out = _tc_final(aggs[0][:N], aggs[2][:N], ci_item,
                       convs[0]["lw"].T, convs[2]["lw"].T,
                       (convs[0]["lb"] + convs[2]["lb"]).reshape(1, D),
                       params["ifc_w"].T, params["ifc_b"].reshape(1, D))
  user_out = _tc_final(aggs[1][:N], aggs[3][:N], ci_user,
                       convs[1]["lw"].T, convs[3]["lw"].T,
                       (convs[1]["lb"] + convs[3]["lb"]).reshape(1, D),
                       params["ufc_w"].T, params["ufc_b"].reshape(1, D))
  return (user_out, item_out)


# speedup vs baseline: 3.5431x; 1.3633x over previous
import functools

import jax
import jax.numpy as jnp
from jax import lax
from jax.experimental import pallas as pl
from jax.experimental.pallas import tpu as pltpu
from jax.experimental.pallas import tpu_sc as plsc

N = 50000
E = 400000
D = 128

NC = 2
NS = 16
NW = NC * NS
CH = 128

EP = ((E + NW * CH - 1) // (NW * CH)) * (NW * CH)
NP = ((N + NW * CH - 1) // (NW * CH)) * (NW * CH)
N1 = 53760
DUMP = N + 1

_MESH = plsc.VectorSubcoreMesh(core_axis_name="c", subcore_axis_name="s")


def _sc_gather(table, idx, d, w=4):
  b = idx.shape[0]
  per_w = b // NW
  nch = per_w // CH
  full, tail = divmod(nch, w)

  @functools.partial(
      pl.kernel,
      out_type=jax.ShapeDtypeStruct((b, d), jnp.float32),
      mesh=_MESH,
      scratch_types=[
          pltpu.VMEM((per_w,), jnp.int32),
          pltpu.VMEM((w * CH, d), jnp.float32),
          pltpu.SemaphoreType.DMA,
          pltpu.SemaphoreType.DMA,
      ],
  )
  def k(table_hbm, idx_hbm, out_hbm, idx_all, rows_v, semg, semw):
    wid = lax.axis_index("s") * NC + lax.axis_index("c")
    base = wid * per_w
    pltpu.sync_copy(idx_hbm.at[pl.ds(base, per_w)], idx_all)

    def do_batch(i0, nb):
      hs = []
      for bb in range(nb):
        hs.append(pltpu.async_copy(
            table_hbm.at[idx_all.at[pl.ds((i0 + bb) * CH, CH)]],
            rows_v.at[pl.ds(bb * CH, CH)], semg))
      for h in hs:
        h.wait()
      hs = []
      for bb in range(nb):
        hs.append(pltpu.async_copy(
            rows_v.at[pl.ds(bb * CH, CH)],
            out_hbm.at[pl.ds(base + (i0 + bb) * CH, CH)], semw))
      for h in hs:
        h.wait()

    def body(i, carry):
      do_batch(i * w, w)
      return carry

    lax.fori_loop(0, full, body, 0)
    if tail:
      do_batch(full * w, tail)

  return k(table, idx)


def _sc_cj_gather(cj, src, w=4):
  per_w = EP // NW
  nch = per_w // CH
  full, tail = divmod(nch, w)

  @functools.partial(
      pl.kernel,
      out_type=jax.ShapeDtypeStruct((EP,), jnp.float32),
      mesh=_MESH,
      scratch_types=[
          pltpu.VMEM((per_w,), jnp.int32),
          pltpu.VMEM((w * CH,), jnp.float32),
          pltpu.SemaphoreType.DMA,
          pltpu.SemaphoreType.DMA,
      ],
  )
  def k(cj_hbm, src_hbm, out_hbm, idx_all, val_v, semg, semw):
    wid = lax.axis_index("s") * NC + lax.axis_index("c")
    base = wid * per_w
    pltpu.sync_copy(src_hbm.at[pl.ds(base, per_w)], idx_all)

    def do_batch(i0, nb):
      hs = []
      for bb in range(nb):
        hs.append(pltpu.async_copy(
            cj_hbm.at[idx_all.at[pl.ds((i0 + bb) * CH, CH)]],
            val_v.at[pl.ds(bb * CH, CH)], semg))
      for h in hs:
        h.wait()
      hs = []
      for bb in range(nb):
        hs.append(pltpu.async_copy(
            val_v.at[pl.ds(bb * CH, CH)],
            out_hbm.at[pl.ds(base + (i0 + bb) * CH, CH)], semw))
      for h in hs:
        h.wait()

    def body(i, carry):
      do_batch(i * w, w)
      return carry

    lax.fori_loop(0, full, body, 0)
    if tail:
      do_batch(full * w, tail)

  return k(cj, src)


def _sc_scatter_add(m, dst):
  e_per_tile = EP // NS
  nch = e_per_tile // CH
  w = 2
  nb = nch // w
  rng = 8960
  arows = 10240
  ldump = arows - 1

  @functools.partial(
      pl.kernel,
      out_type=jax.ShapeDtypeStruct((N1, D), jnp.float32),
      mesh=_MESH,
      scratch_types=[
          pltpu.VMEM((w, CH), jnp.int32),
          pltpu.VMEM((w, CH), jnp.int32),
          pltpu.VMEM((w * CH, D), jnp.float32),
          pltpu.VMEM_SHARED((arows, D), jnp.float32),
          pltpu.SemaphoreType.DMA,
          pltpu.SemaphoreType.DMA,
          pltpu.SemaphoreType.DMA,
      ],
  )
  def k(m_hbm, dst_hbm, out_hbm, idxb, adj, mbuf, agg_sh, semi, sema, semb):
    cid = lax.axis_index("c")
    sid = lax.axis_index("s")
    ebase = sid * e_per_tile
    z16 = jnp.zeros((16,), jnp.float32)

    for third in range(3):
      rp = cid * 3 + third
      lo = rp * rng

      def zbody(r, carry):
        for c0 in range(0, D, 16):
          mbuf[r, c0:c0 + 16] = z16
        return carry

      lax.fori_loop(0, CH, zbody, 0)

      def fill(j, carry):
        pltpu.sync_copy(mbuf.at[pl.ds(0, CH)],
                        agg_sh.at[pl.ds(sid * 640 + j * CH, CH)])
        return carry

      lax.fori_loop(0, 5, fill, 0)
      plsc.subcore_barrier()

      def scat(i, carry):
        hs = []
        for bb in range(w):
          off = ebase + (i * w + bb) * CH
          hs.append(pltpu.async_copy(dst_hbm.at[pl.ds(off, CH)],
                                     idxb.at[bb], semi))
          hs.append(pltpu.async_copy(m_hbm.at[pl.ds(off, CH)],
                                     mbuf.at[pl.ds(bb * CH, CH)], sema))
        for h in hs:
          h.wait()
        for bb in range(w):
          for kk in range(CH // 16):
            v = idxb[bb, kk * 16:(kk + 1) * 16]
            local = v - lo
            ok = (local >= 0) & (local < rng)
            adj[bb, kk * 16:(kk + 1) * 16] = jnp.where(ok, local, ldump)
        hs = []
        for bb in range(w):
          hs.append(pltpu.async_copy(mbuf.at[pl.ds(bb * CH, CH)],
                                     agg_sh.at[adj.at[bb]], semb, add=True))
        for h in hs:
          h.wait()
        return carry

      lax.fori_loop(0, nb, scat, 0)
      plsc.subcore_barrier()

      pltpu.sync_copy(agg_sh.at[pl.ds(sid * 560, 560)],
                      out_hbm.at[pl.ds(lo + sid * 560, 560)])
      plsc.subcore_barrier()

  return k(m, dst)


_NODE_BK = 1000
_EDGE_BK = 512


def _hcj_body(feat_ref, cj_ref, w_ref, b_ref, out_ref):
  h = jnp.dot(feat_ref[...], w_ref[...], preferred_element_type=jnp.float32)
  out_ref[...] = (h + b_ref[...]) * cj_ref[...]


def _tc_hcj(feat, cj, emw_t, emb):
  grid = N // _NODE_BK
  return pl.pallas_call(
      _hcj_body,
      grid=(grid,),
      in_specs=[
          pl.BlockSpec((_NODE_BK, D), lambda i: (i, 0)),
          pl.BlockSpec((_NODE_BK, 1), lambda i: (i, 0)),
          pl.BlockSpec((D, D), lambda i: (0, 0)),
          pl.BlockSpec((1, D), lambda i: (0, 0)),
      ],
      out_specs=pl.BlockSpec((_NODE_BK, D), lambda i: (i, 0)),
      out_shape=jax.ShapeDtypeStruct((N, D), jnp.float32),
  )(feat, cj, emw_t, emb)


def _gelu(x):
  return 0.5 * x * (1.0 + lax.erf(x * 0.7071067811865476))


def _edge_body(rf_ref, hs_ref, cj_ref, pw_ref, rsw_ref,
               w1_ref, w2_ref, w3_ref, out_ref):
  rfeat = rf_ref[...]
  pa = jax.nn.sigmoid(jnp.sum(rfeat * pw_ref[...], axis=1, keepdims=True))
  ra = jax.nn.sigmoid(jnp.sum(rfeat * rsw_ref[...], axis=1, keepdims=True))
  g = _gelu(jnp.dot(rfeat, w1_ref[...], preferred_element_type=jnp.float32))
  g = _gelu(jnp.dot(g, w2_ref[...], preferred_element_type=jnp.float32))
  rf = jnp.dot(g, w3_ref[...], preferred_element_type=jnp.float32)
  cj_row = cj_ref[...].reshape(1, _EDGE_BK)
  rr = lax.broadcasted_iota(jnp.int32, (_EDGE_BK, _EDGE_BK), 0)
  cc = lax.broadcasted_iota(jnp.int32, (_EDGE_BK, _EDGE_BK), 1)
  eye = (rr == cc).astype(jnp.float32)
  cj_col = lax.dot_general(eye, cj_row, (((1,), (1,)), ((), ())),
                           preferred_element_type=jnp.float32)
  out_ref[...] = hs_ref[...] * pa + rf * (ra * cj_col)


def _tc_edge(rfeat, hsrc, cjsrc3, pw, rsw, w1_t, w2_t, w3_t):
  grid = EP // _EDGE_BK
  return pl.pallas_call(
      _edge_body,
      grid=(grid,),
      in_specs=[
          pl.BlockSpec((_EDGE_BK, D), lambda i: (i, 0)),
          pl.BlockSpec((_EDGE_BK, D), lambda i: (i, 0)),
          pl.BlockSpec((1, 1, _EDGE_BK), lambda i: (i, 0, 0)),
          pl.BlockSpec((1, D), lambda i: (0, 0)),
          pl.BlockSpec((1, D), lambda i: (0, 0)),
          pl.BlockSpec((D, D), lambda i: (0, 0)),
          pl.BlockSpec((D, D), lambda i: (0, 0)),
          pl.BlockSpec((D, D), lambda i: (0, 0)),
      ],
      out_specs=pl.BlockSpec((_EDGE_BK, D), lambda i: (i, 0)),
      out_shape=jax.ShapeDtypeStruct((EP, D), jnp.float32),
  )(rfeat, hsrc, cjsrc3, pw, rsw, w1_t, w2_t, w3_t)


def _final_body(a_ref, b_ref, ci_ref, lwa_ref, lwb_ref, lb_ref,
                fc_ref, fcb_ref, out_ref):
  ci = ci_ref[...]
  s = jnp.dot(a_ref[...] * ci, lwa_ref[...], preferred_element_type=jnp.float32)
  s = s + jnp.dot(b_ref[...] * ci, lwb_ref[...],
                  preferred_element_type=jnp.float32)
  s = s + lb_ref[...]
  out_ref[...] = jnp.dot(_gelu(s), fc_ref[...],
                         preferred_element_type=jnp.float32) + fcb_ref[...]


def _tc_final(agg_a, agg_b, ci, lwa_t, lwb_t, lbsum, fc_t, fcb):
  grid = N // _NODE_BK
  return pl.pallas_call(
      _final_body,
      grid=(grid,),
      in_specs=[
          pl.BlockSpec((_NODE_BK, D), lambda i: (i, 0)),
          pl.BlockSpec((_NODE_BK, D), lambda i: (i, 0)),
          pl.BlockSpec((_NODE_BK, 1), lambda i: (i, 0)),
          pl.BlockSpec((D, D), lambda i: (0, 0)),
          pl.BlockSpec((D, D), lambda i: (0, 0)),
          pl.BlockSpec((1, D), lambda i: (0, 0)),
          pl.BlockSpec((D, D), lambda i: (0, 0)),
          pl.BlockSpec((1, D), lambda i: (0, 0)),
      ],
      out_specs=pl.BlockSpec((_NODE_BK, D), lambda i: (i, 0)),
      out_shape=jax.ShapeDtypeStruct((N, D), jnp.float32),
  )(agg_a, agg_b, ci, lwa_t, lwb_t, lbsum, fc_t, fcb)


def _pad_idx(idx, total, fill):
  return jnp.pad(idx.astype(jnp.int32), (0, total - idx.shape[0]),
                 constant_values=fill)


def kernel(input_user, input_item,
           src_fwd_0, dst_fwd_0, rid_fwd_0, src_rev_0, dst_rev_0, rid_rev_0,
           src_fwd_1, dst_fwd_1, rid_fwd_1, src_rev_1, dst_rev_1, rid_rev_1,
           cj_user, ci_user, cj_item, ci_item, params):
  table = params["table"]
  convs = params["convs"]

  feat_u = _sc_gather(params["uemb"], _pad_idx(input_user, NP, 0), D)
  feat_i = _sc_gather(params["iemb"], _pad_idx(input_item, NP, 0), D)

  edges = [
      (src_fwd_0, dst_fwd_0, rid_fwd_0, feat_u, cj_user),
      (src_rev_0, dst_rev_0, rid_rev_0, feat_i, cj_item),
      (src_fwd_1, dst_fwd_1, rid_fwd_1, feat_u, cj_user),
      (src_rev_1, dst_rev_1, rid_rev_1, feat_i, cj_item),
  ]

  aggs = []
  for c, (src, dst, rid, feat, cj) in enumerate(edges):
    cp = convs[c]
    hcj = _tc_hcj(feat, cj, cp["emw"].T, cp["emb"].reshape(1, D))
    src_p = _pad_idx(src, EP, 0)
    rfeat = _sc_gather(table, _pad_idx(rid, EP, 0), D)
    hsrc = _sc_gather(hcj, src_p, D)
    cjsrc = _sc_cj_gather(cj.reshape(N), src_p)
    m = _tc_edge(rfeat, hsrc, cjsrc.reshape(EP // _EDGE_BK, 1, _EDGE_BK),
                 cp["pw"], cp["rsw"],
                 cp["rw1"].T, cp["rw2"].T, cp["rw3"].T)
    agg = _sc_scatter_add(m, _pad_idx(dst, EP, DUMP))
    aggs.append(agg)

  item_out = _tc_final(aggs[0][:N], aggs[2][:N], ci_item,
                       convs[0]["lw"].T, convs[2]["lw"].T,
                       (convs[0]["lb"] + convs[2]["lb"]).reshape(1, D),
                       params["ifc_w"].T, params["ifc_b"].reshape(1, D))
  user_out = _tc_final(aggs[1][:N], aggs[3][:N], ci_user,
                       convs[1]["lw"].T, convs[3]["lw"].T,
                       (convs[1]["lb"] + convs[3]["lb"]).reshape(1, D),
                       params["ufc_w"].T, params["ufc_b"].reshape(1, D))
  return (user_out, item_out)


# speedup vs baseline: 3.5938x; 1.0143x over previous
import functools

import jax
import jax.numpy as jnp
from jax import lax
from jax.experimental import pallas as pl
from jax.experimental.pallas import tpu as pltpu
from jax.experimental.pallas import tpu_sc as plsc

N = 50000
E = 400000
D = 128

NC = 2
NS = 16
NW = NC * NS
CH = 128

EP = ((E + NW * CH - 1) // (NW * CH)) * (NW * CH)
NP = ((N + NW * CH - 1) // (NW * CH)) * (NW * CH)
N1 = 53760
DUMP = N + 1

_MESH = plsc.VectorSubcoreMesh(core_axis_name="c", subcore_axis_name="s")


def _sc_gather(table, idx, d, w=6):
  b = idx.shape[0]
  per_w = b // NW
  nch = per_w // CH
  full, tail = divmod(nch, w)

  @functools.partial(
      pl.kernel,
      out_type=jax.ShapeDtypeStruct((b, d), jnp.float32),
      mesh=_MESH,
      scratch_types=[
          pltpu.VMEM((per_w,), jnp.int32),
          pltpu.VMEM((w * CH, d), jnp.float32),
          pltpu.SemaphoreType.DMA,
          pltpu.SemaphoreType.DMA,
      ],
  )
  def k(table_hbm, idx_hbm, out_hbm, idx_all, rows_v, semg, semw):
    wid = lax.axis_index("s") * NC + lax.axis_index("c")
    base = wid * per_w
    pltpu.sync_copy(idx_hbm.at[pl.ds(base, per_w)], idx_all)

    def do_batch(i0, nb):
      hs = []
      for bb in range(nb):
        hs.append(pltpu.async_copy(
            table_hbm.at[idx_all.at[pl.ds((i0 + bb) * CH, CH)]],
            rows_v.at[pl.ds(bb * CH, CH)], semg))
      for h in hs:
        h.wait()
      hs = []
      for bb in range(nb):
        hs.append(pltpu.async_copy(
            rows_v.at[pl.ds(bb * CH, CH)],
            out_hbm.at[pl.ds(base + (i0 + bb) * CH, CH)], semw))
      for h in hs:
        h.wait()

    def body(i, carry):
      do_batch(i * w, w)
      return carry

    lax.fori_loop(0, full, body, 0)
    if tail:
      do_batch(full * w, tail)

  return k(table, idx)


def _sc_cj_gather(cj, src, w=8):
  per_w = EP // NW
  nch = per_w // CH
  full, tail = divmod(nch, w)

  @functools.partial(
      pl.kernel,
      out_type=jax.ShapeDtypeStruct((EP,), jnp.float32),
      mesh=_MESH,
      scratch_types=[
          pltpu.VMEM((per_w,), jnp.int32),
          pltpu.VMEM((w * CH,), jnp.float32),
          pltpu.SemaphoreType.DMA,
          pltpu.SemaphoreType.DMA,
      ],
  )
  def k(cj_hbm, src_hbm, out_hbm, idx_all, val_v, semg, semw):
    wid = lax.axis_index("s") * NC + lax.axis_index("c")
    base = wid * per_w
    pltpu.sync_copy(src_hbm.at[pl.ds(base, per_w)], idx_all)

    def do_batch(i0, nb):
      hs = []
      for bb in range(nb):
        hs.append(pltpu.async_copy(
            cj_hbm.at[idx_all.at[pl.ds((i0 + bb) * CH, CH)]],
            val_v.at[pl.ds(bb * CH, CH)], semg))
      for h in hs:
        h.wait()
      hs = []
      for bb in range(nb):
        hs.append(pltpu.async_copy(
            val_v.at[pl.ds(bb * CH, CH)],
            out_hbm.at[pl.ds(base + (i0 + bb) * CH, CH)], semw))
      for h in hs:
        h.wait()

    def body(i, carry):
      do_batch(i * w, w)
      return carry

    lax.fori_loop(0, full, body, 0)
    if tail:
      do_batch(full * w, tail)

  return k(cj, src)


def _sc_scatter_add(m, dst):
  e_per_tile = EP // NS
  nch = e_per_tile // CH
  w = 2
  nb = nch // w
  rng = 8960
  arows = 10240
  ldump = arows - 1

  @functools.partial(
      pl.kernel,
      out_type=jax.ShapeDtypeStruct((N1, D), jnp.float32),
      mesh=_MESH,
      scratch_types=[
          pltpu.VMEM((w, CH), jnp.int32),
          pltpu.VMEM((w, CH), jnp.int32),
          pltpu.VMEM((w * CH, D), jnp.float32),
          pltpu.VMEM_SHARED((arows, D), jnp.float32),
          pltpu.SemaphoreType.DMA,
          pltpu.SemaphoreType.DMA,
          pltpu.SemaphoreType.DMA,
      ],
  )
  def k(m_hbm, dst_hbm, out_hbm, idxb, adj, mbuf, agg_sh, semi, sema, semb):
    cid = lax.axis_index("c")
    sid = lax.axis_index("s")
    ebase = sid * e_per_tile
    z16 = jnp.zeros((16,), jnp.float32)

    for third in range(3):
      rp = cid * 3 + third
      lo = rp * rng

      def zbody(r, carry):
        for c0 in range(0, D, 16):
          mbuf[r, c0:c0 + 16] = z16
        return carry

      lax.fori_loop(0, CH, zbody, 0)

      def fill(j, carry):
        pltpu.sync_copy(mbuf.at[pl.ds(0, CH)],
                        agg_sh.at[pl.ds(sid * 640 + j * CH, CH)])
        return carry

      lax.fori_loop(0, 5, fill, 0)
      plsc.subcore_barrier()

      def scat(i, carry):
        hs = []
        for bb in range(w):
          off = ebase + (i * w + bb) * CH
          hs.append(pltpu.async_copy(dst_hbm.at[pl.ds(off, CH)],
                                     idxb.at[bb], semi))
          hs.append(pltpu.async_copy(m_hbm.at[pl.ds(off, CH)],
                                     mbuf.at[pl.ds(bb * CH, CH)], sema))
        for h in hs:
          h.wait()
        for bb in range(w):
          for kk in range(CH // 16):
            v = idxb[bb, kk * 16:(kk + 1) * 16]
            local = v - lo
            ok = (local >= 0) & (local < rng)
            adj[bb, kk * 16:(kk + 1) * 16] = jnp.where(ok, local, ldump)
        hs = []
        for bb in range(w):
          hs.append(pltpu.async_copy(mbuf.at[pl.ds(bb * CH, CH)],
                                     agg_sh.at[adj.at[bb]], semb, add=True))
        for h in hs:
          h.wait()
        return carry

      lax.fori_loop(0, nb, scat, 0)
      plsc.subcore_barrier()

      pltpu.sync_copy(agg_sh.at[pl.ds(sid * 560, 560)],
                      out_hbm.at[pl.ds(lo + sid * 560, 560)])
      plsc.subcore_barrier()

  return k(m, dst)


_NODE_BK = 1000
_EDGE_BK = 512


def _hcj_body(feat_ref, cj_ref, w_ref, b_ref, out_ref):
  h = jnp.dot(feat_ref[...], w_ref[...], preferred_element_type=jnp.float32)
  out_ref[...] = (h + b_ref[...]) * cj_ref[...]


def _tc_hcj(feat, cj, emw_t, emb):
  grid = N // _NODE_BK
  return pl.pallas_call(
      _hcj_body,
      grid=(grid,),
      in_specs=[
          pl.BlockSpec((_NODE_BK, D), lambda i: (i, 0)),
          pl.BlockSpec((_NODE_BK, 1), lambda i: (i, 0)),
          pl.BlockSpec((D, D), lambda i: (0, 0)),
          pl.BlockSpec((1, D), lambda i: (0, 0)),
      ],
      out_specs=pl.BlockSpec((_NODE_BK, D), lambda i: (i, 0)),
      out_shape=jax.ShapeDtypeStruct((N, D), jnp.float32),
  )(feat, cj, emw_t, emb)


def _gelu(x):
  return 0.5 * x * (1.0 + lax.erf(x * 0.7071067811865476))


def _edge_body(rf_ref, hs_ref, cj_ref, pw_ref, rsw_ref,
               w1_ref, w2_ref, w3_ref, out_ref):
  rfeat = rf_ref[...]
  pa = jax.nn.sigmoid(jnp.sum(rfeat * pw_ref[...], axis=1, keepdims=True))
  ra = jax.nn.sigmoid(jnp.sum(rfeat * rsw_ref[...], axis=1, keepdims=True))
  g = _gelu(jnp.dot(rfeat, w1_ref[...], preferred_element_type=jnp.float32))
  g = _gelu(jnp.dot(g, w2_ref[...], preferred_element_type=jnp.float32))
  rf = jnp.dot(g, w3_ref[...], preferred_element_type=jnp.float32)
  cj_row = cj_ref[...].reshape(1, _EDGE_BK)
  rr = lax.broadcasted_iota(jnp.int32, (_EDGE_BK, _EDGE_BK), 0)
  cc = lax.broadcasted_iota(jnp.int32, (_EDGE_BK, _EDGE_BK), 1)
  eye = (rr == cc).astype(jnp.float32)
  cj_col = lax.dot_general(eye, cj_row, (((1,), (1,)), ((), ())),
                           preferred_element_type=jnp.float32)
  out_ref[...] = hs_ref[...] * pa + rf * (ra * cj_col)


def _tc_edge(rfeat, hsrc, cjsrc3, pw, rsw, w1_t, w2_t, w3_t):
  grid = EP // _EDGE_BK
  return pl.pallas_call(
      _edge_body,
      grid=(grid,),
      in_specs=[
          pl.BlockSpec((_EDGE_BK, D), lambda i: (i, 0)),
          pl.BlockSpec((_EDGE_BK, D), lambda i: (i, 0)),
          pl.BlockSpec((1, 1, _EDGE_BK), lambda i: (i, 0, 0)),
          pl.BlockSpec((1, D), lambda i: (0, 0)),
          pl.BlockSpec((1, D), lambda i: (0, 0)),
          pl.BlockSpec((D, D), lambda i: (0, 0)),
          pl.BlockSpec((D, D), lambda i: (0, 0)),
          pl.BlockSpec((D, D), lambda i: (0, 0)),
      ],
      out_specs=pl.BlockSpec((_EDGE_BK, D), lambda i: (i, 0)),
      out_shape=jax.ShapeDtypeStruct((EP, D), jnp.float32),
  )(rfeat, hsrc, cjsrc3, pw, rsw, w1_t, w2_t, w3_t)


def _final_body(a_ref, b_ref, ci_ref, lwa_ref, lwb_ref, lb_ref,
                fc_ref, fcb_ref, out_ref):
  ci = ci_ref[...]
  s = jnp.dot(a_ref[...] * ci, lwa_ref[...], preferred_element_type=jnp.float32)
  s = s + jnp.dot(b_ref[...] * ci, lwb_ref[...],
                  preferred_element_type=jnp.float32)
  s = s + lb_ref[...]
  out_ref[...] = jnp.dot(_gelu(s), fc_ref[...],
                         preferred_element_type=jnp.float32) + fcb_ref[...]


def _tc_final(agg_a, agg_b, ci, lwa_t, lwb_t, lbsum, fc_t, fcb):
  grid = N // _NODE_BK
  return pl.pallas_call(
      _final_body,
      grid=(grid,),
      in_specs=[
          pl.BlockSpec((_NODE_BK, D), lambda i: (i, 0)),
          pl.BlockSpec((_NODE_BK, D), lambda i: (i, 0)),
          pl.BlockSpec((_NODE_BK, 1), lambda i: (i, 0)),
          pl.BlockSpec((D, D), lambda i: (0, 0)),
          pl.BlockSpec((D, D), lambda i: (0, 0)),
          pl.BlockSpec((1, D), lambda i: (0, 0)),
          pl.BlockSpec((D, D), lambda i: (0, 0)),
          pl.BlockSpec((1, D), lambda i: (0, 0)),
      ],
      out_specs=pl.BlockSpec((_NODE_BK, D), lambda i: (i, 0)),
      out_shape=jax.ShapeDtypeStruct((N, D), jnp.float32),
  )(agg_a, agg_b, ci, lwa_t, lwb_t, lbsum, fc_t, fcb)


def _pad_idx(idx, total, fill):
  return jnp.pad(idx.astype(jnp.int32), (0, total - idx.shape[0]),
                 constant_values=fill)


def kernel(input_user, input_item,
           src_fwd_0, dst_fwd_0, rid_fwd_0, src_rev_0, dst_rev_0, rid_rev_0,
           src_fwd_1, dst_fwd_1, rid_fwd_1, src_rev_1, dst_rev_1, rid_rev_1,
           cj_user, ci_user, cj_item, ci_item, params):
  table = params["table"]
  convs = params["convs"]

  feat_u = _sc_gather(params["uemb"], _pad_idx(input_user, NP, 0), D)
  feat_i = _sc_gather(params["iemb"], _pad_idx(input_item, NP, 0), D)

  edges = [
      (src_fwd_0, dst_fwd_0, rid_fwd_0, feat_u, cj_user),
      (src_rev_0, dst_rev_0, rid_rev_0, feat_i, cj_item),
      (src_fwd_1, dst_fwd_1, rid_fwd_1, feat_u, cj_user),
      (src_rev_1, dst_rev_1, rid_rev_1, feat_i, cj_item),
  ]

  aggs = []
  for c, (src, dst, rid, feat, cj) in enumerate(edges):
    cp = convs[c]
    hcj = _tc_hcj(feat, cj, cp["emw"].T, cp["emb"].reshape(1, D))
    src_p = _pad_idx(src, EP, 0)
    rfeat = _sc_gather(table, _pad_idx(rid, EP, 0), D)
    hsrc = _sc_gather(hcj, src_p, D)
    cjsrc = _sc_cj_gather(cj.reshape(N), src_p)
    m = _tc_edge(rfeat, hsrc, cjsrc.reshape(EP // _EDGE_BK, 1, _EDGE_BK),
                 cp["pw"], cp["rsw"],
                 cp["rw1"].T, cp["rw2"].T, cp["rw3"].T)
    agg = _sc_scatter_add(m, _pad_idx(dst, EP, DUMP))
    aggs.append(agg)

  item_out = _tc_final(aggs[0][:N], aggs[2][:N], ci_item,
                       convs[0]["lw"].T, convs[2]["lw"].T,
                       (convs[0]["lb"] + convs[2]["lb"]).reshape(1, D),
                       params["ifc_w"].T, params["ifc_b"].reshape(1, D))
  user_out = _tc_final(aggs[1][:N], aggs[3][:N], ci_user,
                       convs[1]["lw"].T, convs[3]["lw"].T,
                       (convs[1]["lb"] + convs[3]["lb"]).reshape(1, D),
                       params["ufc_w"].T, params["ufc_b"].reshape(1, D))
  return (user_out, item_out)


# speedup vs baseline: 4.6224x; 1.2862x over previous
import functools

import jax
import jax.numpy as jnp
from jax import lax
from jax.experimental import pallas as pl
from jax.experimental.pallas import tpu as pltpu
from jax.experimental.pallas import tpu_sc as plsc

N = 50000
E = 400000
D = 128

NC = 2
NS = 16
NW = NC * NS
CH = 128

EP = ((E + NW * CH - 1) // (NW * CH)) * (NW * CH)
NP = ((N + NW * CH - 1) // (NW * CH)) * (NW * CH)
N1 = 50176
DUMP = N + 1

_MESH = plsc.VectorSubcoreMesh(core_axis_name="c", subcore_axis_name="s")


def _sc_gather(table, idx, d, w=6):
  b = idx.shape[0]
  per_w = b // NW
  nch = per_w // CH
  full, tail = divmod(nch, w)

  @functools.partial(
      pl.kernel,
      out_type=jax.ShapeDtypeStruct((b, d), jnp.float32),
      mesh=_MESH,
      scratch_types=[
          pltpu.VMEM((per_w,), jnp.int32),
          pltpu.VMEM((w * CH, d), jnp.float32),
          pltpu.SemaphoreType.DMA,
          pltpu.SemaphoreType.DMA,
      ],
  )
  def k(table_hbm, idx_hbm, out_hbm, idx_all, rows_v, semg, semw):
    wid = lax.axis_index("s") * NC + lax.axis_index("c")
    base = wid * per_w
    pltpu.sync_copy(idx_hbm.at[pl.ds(base, per_w)], idx_all)

    def do_batch(i0, nb):
      hs = []
      for bb in range(nb):
        hs.append(pltpu.async_copy(
            table_hbm.at[idx_all.at[pl.ds((i0 + bb) * CH, CH)]],
            rows_v.at[pl.ds(bb * CH, CH)], semg))
      for h in hs:
        h.wait()
      hs = []
      for bb in range(nb):
        hs.append(pltpu.async_copy(
            rows_v.at[pl.ds(bb * CH, CH)],
            out_hbm.at[pl.ds(base + (i0 + bb) * CH, CH)], semw))
      for h in hs:
        h.wait()

    def body(i, carry):
      do_batch(i * w, w)
      return carry

    lax.fori_loop(0, full, body, 0)
    if tail:
      do_batch(full * w, tail)

  return k(table, idx)


def _sc_cj_gather(cj, src, w=8):
  per_w = EP // NW
  nch = per_w // CH
  full, tail = divmod(nch, w)

  @functools.partial(
      pl.kernel,
      out_type=jax.ShapeDtypeStruct((EP,), jnp.float32),
      mesh=_MESH,
      scratch_types=[
          pltpu.VMEM((per_w,), jnp.int32),
          pltpu.VMEM((w * CH,), jnp.float32),
          pltpu.SemaphoreType.DMA,
          pltpu.SemaphoreType.DMA,
      ],
  )
  def k(cj_hbm, src_hbm, out_hbm, idx_all, val_v, semg, semw):
    wid = lax.axis_index("s") * NC + lax.axis_index("c")
    base = wid * per_w
    pltpu.sync_copy(src_hbm.at[pl.ds(base, per_w)], idx_all)

    def do_batch(i0, nb):
      hs = []
      for bb in range(nb):
        hs.append(pltpu.async_copy(
            cj_hbm.at[idx_all.at[pl.ds((i0 + bb) * CH, CH)]],
            val_v.at[pl.ds(bb * CH, CH)], semg))
      for h in hs:
        h.wait()
      hs = []
      for bb in range(nb):
        hs.append(pltpu.async_copy(
            val_v.at[pl.ds(bb * CH, CH)],
            out_hbm.at[pl.ds(base + (i0 + bb) * CH, CH)], semw))
      for h in hs:
        h.wait()

    def body(i, carry):
      do_batch(i * w, w)
      return carry

    lax.fori_loop(0, full, body, 0)
    if tail:
      do_batch(full * w, tail)

  return k(cj, src)


def _sc_scatter_add(m, dst):
  e_per_tile = EP // NS
  nch = e_per_tile // CH
  rng = 12544
  arows = 12552
  ldump = rng

  @functools.partial(
      pl.kernel,
      out_type=jax.ShapeDtypeStruct((N1, D), jnp.float32),
      mesh=_MESH,
      scratch_types=[
          pltpu.VMEM((1, CH), jnp.int32),
          pltpu.VMEM((1, CH), jnp.int32),
          pltpu.VMEM((CH, D), jnp.float32),
          pltpu.VMEM_SHARED((arows, D), jnp.float32),
          pltpu.SemaphoreType.DMA,
          pltpu.SemaphoreType.DMA,
          pltpu.SemaphoreType.DMA,
      ],
  )
  def k(m_hbm, dst_hbm, out_hbm, idxb, adj, mbuf, agg_sh, semi, sema, semb):
    cid = lax.axis_index("c")
    sid = lax.axis_index("s")
    ebase = sid * e_per_tile
    z16 = jnp.zeros((16,), jnp.float32)

    for half in range(2):
      rp = cid * 2 + half
      lo = rp * rng

      def zbody(r, carry):
        for c0 in range(0, D, 16):
          mbuf[r, c0:c0 + 16] = z16
        return carry

      lax.fori_loop(0, CH, zbody, 0)

      def fill(j, carry):
        pltpu.sync_copy(mbuf.at[pl.ds(0, CH)],
                        agg_sh.at[pl.ds(sid * 784 + j * CH, CH)])
        return carry

      lax.fori_loop(0, 6, fill, 0)
      pltpu.sync_copy(mbuf.at[pl.ds(0, 16)],
                      agg_sh.at[pl.ds(sid * 784 + 768, 16)])
      pltpu.sync_copy(mbuf.at[pl.ds(0, 8)], agg_sh.at[pl.ds(ldump, 8)])
      plsc.subcore_barrier()

      def scat(i, carry):
        off = ebase + i * CH
        h1 = pltpu.async_copy(dst_hbm.at[pl.ds(off, CH)], idxb.at[0], semi)
        h2 = pltpu.async_copy(m_hbm.at[pl.ds(off, CH)], mbuf, sema)
        h1.wait()
        h2.wait()
        for kk in range(CH // 16):
          v = idxb[0, kk * 16:(kk + 1) * 16]
          local = v - lo
          ok = (local >= 0) & (local < rng)
          adj[0, kk * 16:(kk + 1) * 16] = jnp.where(ok, local, ldump)
        pltpu.async_copy(mbuf, agg_sh.at[adj.at[0]], semb, add=True).wait()
        return carry

      lax.fori_loop(0, nch, scat, 0)
      plsc.subcore_barrier()

      pltpu.sync_copy(agg_sh.at[pl.ds(sid * 784, 784)],
                      out_hbm.at[pl.ds(lo + sid * 784, 784)])
      plsc.subcore_barrier()

  return k(m, dst)


_NODE_BK = 1000
_EDGE_BK = 512


def _hcj_body(feat_ref, cj_ref, w_ref, b_ref, out_ref):
  h = jnp.dot(feat_ref[...], w_ref[...], preferred_element_type=jnp.float32)
  out_ref[...] = (h + b_ref[...]) * cj_ref[...]


def _tc_hcj(feat, cj, emw_t, emb):
  grid = N // _NODE_BK
  return pl.pallas_call(
      _hcj_body,
      grid=(grid,),
      in_specs=[
          pl.BlockSpec((_NODE_BK, D), lambda i: (i, 0)),
          pl.BlockSpec((_NODE_BK, 1), lambda i: (i, 0)),
          pl.BlockSpec((D, D), lambda i: (0, 0)),
          pl.BlockSpec((1, D), lambda i: (0, 0)),
      ],
      out_specs=pl.BlockSpec((_NODE_BK, D), lambda i: (i, 0)),
      out_shape=jax.ShapeDtypeStruct((N, D), jnp.float32),
  )(feat, cj, emw_t, emb)


def _gelu(x):
  return 0.5 * x * (1.0 + lax.erf(x * 0.7071067811865476))


def _edge_body(rf_ref, hs_ref, cj_ref, pw_ref, rsw_ref,
               w1_ref, w2_ref, w3_ref, out_ref):
  rfeat = rf_ref[...]
  pa = jax.nn.sigmoid(jnp.sum(rfeat * pw_ref[...], axis=1, keepdims=True))
  ra = jax.nn.sigmoid(jnp.sum(rfeat * rsw_ref[...], axis=1, keepdims=True))
  g = _gelu(jnp.dot(rfeat, w1_ref[...], preferred_element_type=jnp.float32))
  g = _gelu(jnp.dot(g, w2_ref[...], preferred_element_type=jnp.float32))
  rf = jnp.dot(g, w3_ref[...], preferred_element_type=jnp.float32)
  cj_row = cj_ref[...].reshape(1, _EDGE_BK)
  rr = lax.broadcasted_iota(jnp.int32, (_EDGE_BK, _EDGE_BK), 0)
  cc = lax.broadcasted_iota(jnp.int32, (_EDGE_BK, _EDGE_BK), 1)
  eye = (rr == cc).astype(jnp.float32)
  cj_col = lax.dot_general(eye, cj_row, (((1,), (1,)), ((), ())),
                           preferred_element_type=jnp.float32)
  out_ref[...] = hs_ref[...] * pa + rf * (ra * cj_col)


def _tc_edge(rfeat, hsrc, cjsrc3, pw, rsw, w1_t, w2_t, w3_t):
  grid = EP // _EDGE_BK
  return pl.pallas_call(
      _edge_body,
      grid=(grid,),
      in_specs=[
          pl.BlockSpec((_EDGE_BK, D), lambda i: (i, 0)),
          pl.BlockSpec((_EDGE_BK, D), lambda i: (i, 0)),
          pl.BlockSpec((1, 1, _EDGE_BK), lambda i: (i, 0, 0)),
          pl.BlockSpec((1, D), lambda i: (0, 0)),
          pl.BlockSpec((1, D), lambda i: (0, 0)),
          pl.BlockSpec((D, D), lambda i: (0, 0)),
          pl.BlockSpec((D, D), lambda i: (0, 0)),
          pl.BlockSpec((D, D), lambda i: (0, 0)),
      ],
      out_specs=pl.BlockSpec((_EDGE_BK, D), lambda i: (i, 0)),
      out_shape=jax.ShapeDtypeStruct((EP, D), jnp.float32),
  )(rfeat, hsrc, cjsrc3, pw, rsw, w1_t, w2_t, w3_t)


def _final_body(a_ref, b_ref, ci_ref, lwa_ref, lwb_ref, lb_ref,
                fc_ref, fcb_ref, out_ref):
  ci = ci_ref[...]
  s = jnp.dot(a_ref[...] * ci, lwa_ref[...], preferred_element_type=jnp.float32)
  s = s + jnp.dot(b_ref[...] * ci, lwb_ref[...],
                  preferred_element_type=jnp.float32)
  s = s + lb_ref[...]
  out_ref[...] = jnp.dot(_gelu(s), fc_ref[...],
                         preferred_element_type=jnp.float32) + fcb_ref[...]


def _tc_final(agg_a, agg_b, ci, lwa_t, lwb_t, lbsum, fc_t, fcb):
  grid = N // _NODE_BK
  return pl.pallas_call(
      _final_body,
      grid=(grid,),
      in_specs=[
          pl.BlockSpec((_NODE_BK, D), lambda i: (i, 0)),
          pl.BlockSpec((_NODE_BK, D), lambda i: (i, 0)),
          pl.BlockSpec((_NODE_BK, 1), lambda i: (i, 0)),
          pl.BlockSpec((D, D), lambda i: (0, 0)),
          pl.BlockSpec((D, D), lambda i: (0, 0)),
          pl.BlockSpec((1, D), lambda i: (0, 0)),
          pl.BlockSpec((D, D), lambda i: (0, 0)),
          pl.BlockSpec((1, D), lambda i: (0, 0)),
      ],
      out_specs=pl.BlockSpec((_NODE_BK, D), lambda i: (i, 0)),
      out_shape=jax.ShapeDtypeStruct((N, D), jnp.float32),
  )(agg_a, agg_b, ci, lwa_t, lwb_t, lbsum, fc_t, fcb)


def _pad_idx(idx, total, fill):
  return jnp.pad(idx.astype(jnp.int32), (0, total - idx.shape[0]),
                 constant_values=fill)


def kernel(input_user, input_item,
           src_fwd_0, dst_fwd_0, rid_fwd_0, src_rev_0, dst_rev_0, rid_rev_0,
           src_fwd_1, dst_fwd_1, rid_fwd_1, src_rev_1, dst_rev_1, rid_rev_1,
           cj_user, ci_user, cj_item, ci_item, params):
  table = params["table"]
  convs = params["convs"]

  feat_u = _sc_gather(params["uemb"], _pad_idx(input_user, NP, 0), D)
  feat_i = _sc_gather(params["iemb"], _pad_idx(input_item, NP, 0), D)

  edges = [
      (src_fwd_0, dst_fwd_0, rid_fwd_0, feat_u, cj_user),
      (src_rev_0, dst_rev_0, rid_rev_0, feat_i, cj_item),
      (src_fwd_1, dst_fwd_1, rid_fwd_1, feat_u, cj_user),
      (src_rev_1, dst_rev_1, rid_rev_1, feat_i, cj_item),
  ]

  aggs = []
  for c, (src, dst, rid, feat, cj) in enumerate(edges):
    cp = convs[c]
    hcj = _tc_hcj(feat, cj, cp["emw"].T, cp["emb"].reshape(1, D))
    src_p = _pad_idx(src, EP, 0)
    rfeat = _sc_gather(table, _pad_idx(rid, EP, 0), D)
    hsrc = _sc_gather(hcj, src_p, D)
    cjsrc = _sc_cj_gather(cj.reshape(N), src_p)
    m = _tc_edge(rfeat, hsrc, cjsrc.reshape(EP // _EDGE_BK, 1, _EDGE_BK),
                 cp["pw"], cp["rsw"],
                 cp["rw1"].T, cp["rw2"].T, cp["rw3"].T)
    agg = _sc_scatter_add(m, _pad_idx(dst, EP, DUMP))
    aggs.append(agg)

  item_out = _tc_final(aggs[0][:N], aggs[2][:N], ci_item,
                       convs[0]["lw"].T, convs[2]["lw"].T,
                       (convs[0]["lb"] + convs[2]["lb"]).reshape(1, D),
                       params["ifc_w"].T, params["ifc_b"].reshape(1, D))
  user_out = _tc_final(aggs[1][:N], aggs[3][:N], ci_user,
                       convs[1]["lw"].T, convs[3]["lw"].T,
                       (convs[1]["lb"] + convs[3]["lb"]).reshape(1, D),
                       params["ufc_w"].T, params["ufc_b"].reshape(1, D))
  return (user_out, item_out)
